# baseline (speedup 1.0000x reference)
import functools

import jax
import jax.numpy as jnp
from jax import lax
from jax.experimental import pallas as pl
from jax.experimental.pallas import tpu as pltpu
from jax.experimental.pallas import tpu_sc as plsc

VOCAB = 1000000
EMBED = 64
B = 1024
L = 200
OUT = 2
PAD = 0

NC = 2
NS = 16
NW = NC * NS
ROWS_PER_W = B // NW
LPAD = 208
C0 = 128
C1 = L - C0
NLANE = 16


def _sc_pool(ids_flat, vectors, w):
    mesh = plsc.VectorSubcoreMesh(core_axis_name="c", subcore_axis_name="s")

    @functools.partial(
        pl.kernel,
        out_type=(
            jax.ShapeDtypeStruct((B, EMBED), jnp.float32),
            jax.ShapeDtypeStruct((B, NLANE), jnp.float32),
        ),
        mesh=mesh,
        compiler_params=pltpu.CompilerParams(use_tc_tiling_on_sc=False),
        scratch_types=[
            pltpu.VMEM((LPAD,), jnp.int32),
            pltpu.VMEM((LPAD,), jnp.float32),
            pltpu.VMEM((L, EMBED), jnp.float32),
            pltpu.VMEM((LPAD,), jnp.float32),
            pltpu.VMEM((ROWS_PER_W, EMBED), jnp.float32),
            pltpu.VMEM((ROWS_PER_W, NLANE), jnp.float32),
            pltpu.SemaphoreType.DMA,
        ],
    )
    def k(ids_hbm, vec_hbm, w_hbm, pooled_hbm, len_hbm,
          idx_v, wv_v, rows_v, wt_v, pooled_v, len_v, sem):
        wid = lax.axis_index("s") * NC + lax.axis_index("c")
        row0 = wid * ROWS_PER_W
        lanes = lax.iota(jnp.int32, NLANE)

        def body(i, _):
            row = row0 + i
            base = pl.multiple_of(row * L, 8)
            pltpu.sync_copy(ids_hbm.at[pl.ds(base, L)], idx_v.at[pl.ds(0, L)])
            cps = (
                pltpu.async_copy(vec_hbm.at[idx_v.at[pl.ds(0, C0)]],
                                 rows_v.at[pl.ds(0, C0)], sem),
                pltpu.async_copy(vec_hbm.at[idx_v.at[pl.ds(C0, C1)]],
                                 rows_v.at[pl.ds(C0, C1)], sem),
                pltpu.async_copy(w_hbm.at[idx_v.at[pl.ds(0, C0)]],
                                 wv_v.at[pl.ds(0, C0)], sem),
                pltpu.async_copy(w_hbm.at[idx_v.at[pl.ds(C0, C1)]],
                                 wv_v.at[pl.ds(C0, C1)], sem),
            )
            for cp in cps:
                cp.wait()

            cnt = jnp.zeros((NLANE,), jnp.float32)
            for c in range(LPAD // NLANE):
                ids_c = idx_v[pl.ds(c * NLANE, NLANE)]
                wv_c = wv_v[pl.ds(c * NLANE, NLANE)]
                m = jnp.logical_and(lanes + (c * NLANE) < L, ids_c != PAD)
                sig = 1.0 / (1.0 + jnp.exp(-wv_c))
                wt_v[pl.ds(c * NLANE, NLANE)] = jnp.where(m, sig, 0.0)
                cnt = cnt + jnp.where(m, 1.0, 0.0)
            len_v[i, pl.ds(0, NLANE)] = cnt

            def addto(accs, l, s):
                return tuple(accs[k] + s * rows_v[l, pl.ds(k * NLANE, NLANE)]
                             for k in range(EMBED // NLANE))

            def group_body(g, accs):
                gbase = pl.multiple_of(g * NLANE, NLANE)
                wtg = wt_v[pl.ds(gbase, NLANE)]
                for j in range(NLANE):
                    accs = addto(accs, gbase + j, wtg[j])
                return accs

            accs = lax.fori_loop(
                0, L // NLANE, group_body,
                tuple(jnp.zeros((NLANE,), jnp.float32)
                      for _ in range(EMBED // NLANE)))
            wtg = wt_v[pl.ds((L // NLANE) * NLANE, NLANE)]
            for j in range(L % NLANE):
                accs = addto(accs, (L // NLANE) * NLANE + j, wtg[j])
            for j in range(EMBED // NLANE):
                pooled_v[i, pl.ds(j * NLANE, NLANE)] = accs[j]
            return 0

        lax.fori_loop(0, ROWS_PER_W, body, 0)
        pltpu.sync_copy(pooled_v, pooled_hbm.at[pl.ds(row0, ROWS_PER_W)])
        pltpu.sync_copy(len_v, len_hbm.at[pl.ds(row0, ROWS_PER_W)])

    return k(ids_flat, vectors, w)


def _head(pooled, length, head_W, head_b):
    def hk(p_ref, l_ref, w_ref, b_ref, log_ref, enc_ref):
        length = jnp.sum(l_ref[...], axis=1, keepdims=True) + 1e-16
        p = p_ref[...] / length
        norm = jnp.sqrt(jnp.sum(p * p, axis=1, keepdims=True))
        enc = p / jnp.maximum(norm, 1e-12)
        enc_ref[...] = enc
        log_ref[...] = (
            jnp.dot(enc, w_ref[...], preferred_element_type=jnp.float32)
            + b_ref[...])

    return pl.pallas_call(
        hk,
        out_shape=(
            jax.ShapeDtypeStruct((B, OUT), jnp.float32),
            jax.ShapeDtypeStruct((B, EMBED), jnp.float32),
        ),
    )(pooled, length, head_W, head_b)


def kernel(input_ids, vectors, w, head_W, head_b):
    ids_flat = input_ids.reshape(-1).astype(jnp.int32)
    pooled, counts = _sc_pool(ids_flat, vectors, w)
    logits, encoded = _head(pooled, counts, head_W, head_b.reshape(1, OUT))
    return (logits, encoded)


# speedup vs baseline: 1.2032x; 1.2032x over previous
import functools

import jax
import jax.numpy as jnp
from jax import lax
from jax.experimental import pallas as pl
from jax.experimental.pallas import tpu as pltpu
from jax.experimental.pallas import tpu_sc as plsc

VOCAB = 1000000
EMBED = 64
B = 1024
L = 200
OUT = 2
PAD = 0

NC = 2
NS = 16
NW = NC * NS
ROWS_PER_W = B // NW
LPAD = 208
C0 = 128
C1 = L - C0
NLANE = 16
PAIR = 2 * EMBED
VPAIR = VOCAB // 2
TW = 2048


def _row_table(vt):
    def tk(vt_ref, out_ref):
        x = vt_ref[...]
        out_ref[:, 0:EMBED] = jnp.transpose(x)

    grid = (VOCAB + TW - 1) // TW
    return pl.pallas_call(
        tk,
        grid=(grid,),
        in_specs=[pl.BlockSpec((EMBED, TW), lambda j: (0, j))],
        out_specs=pl.BlockSpec((TW, PAIR), lambda j: (j, 0)),
        out_shape=jax.ShapeDtypeStruct((VOCAB, PAIR), jnp.float32),
    )(vt)


def _sc_pool(ids_flat, pairs, w):
    mesh = plsc.VectorSubcoreMesh(core_axis_name="c", subcore_axis_name="s")

    @functools.partial(
        pl.kernel,
        out_type=(
            jax.ShapeDtypeStruct((B, EMBED), jnp.float32),
            jax.ShapeDtypeStruct((B, NLANE), jnp.float32),
        ),
        mesh=mesh,
        compiler_params=pltpu.CompilerParams(use_tc_tiling_on_sc=True),
        scratch_types=[
            pltpu.VMEM((LPAD,), jnp.int32),
            pltpu.VMEM((LPAD,), jnp.float32),
            pltpu.VMEM((L, PAIR), jnp.float32),
            pltpu.VMEM((LPAD,), jnp.float32),
            pltpu.VMEM((ROWS_PER_W, EMBED), jnp.float32),
            pltpu.VMEM((ROWS_PER_W, NLANE), jnp.float32),
            pltpu.SemaphoreType.DMA,
        ],
    )
    def k(ids_hbm, pairs_hbm, w_hbm, pooled_hbm, len_hbm,
          idx_v, wv_v, rows_v, wt_v, pooled_v, len_v, sem):
        wid = lax.axis_index("s") * NC + lax.axis_index("c")
        row0 = wid * ROWS_PER_W
        lanes = lax.iota(jnp.int32, NLANE)

        def body(i, _):
            row = row0 + i
            base = pl.multiple_of(row * L, 8)
            pltpu.sync_copy(ids_hbm.at[pl.ds(base, L)], idx_v.at[pl.ds(0, L)])
            cps = (
                pltpu.async_copy(pairs_hbm.at[idx_v.at[pl.ds(0, C0)]],
                                 rows_v.at[pl.ds(0, C0)], sem),
                pltpu.async_copy(pairs_hbm.at[idx_v.at[pl.ds(C0, C1)]],
                                 rows_v.at[pl.ds(C0, C1)], sem),
                pltpu.async_copy(w_hbm.at[idx_v.at[pl.ds(0, C0)]],
                                 wv_v.at[pl.ds(0, C0)], sem),
                pltpu.async_copy(w_hbm.at[idx_v.at[pl.ds(C0, C1)]],
                                 wv_v.at[pl.ds(C0, C1)], sem),
            )
            for cp in cps:
                cp.wait()

            cnt = jnp.zeros((NLANE,), jnp.float32)
            for c in range(LPAD // NLANE):
                ids_c = idx_v[pl.ds(c * NLANE, NLANE)]
                wv_c = wv_v[pl.ds(c * NLANE, NLANE)]
                m = jnp.logical_and(lanes + (c * NLANE) < L, ids_c != PAD)
                sig = 1.0 / (1.0 + jnp.exp(-wv_c))
                wt_v[pl.ds(c * NLANE, NLANE)] = jnp.where(m, sig, 0.0)
                cnt = cnt + jnp.where(m, 1.0, 0.0)
            len_v[i, pl.ds(0, NLANE)] = cnt

            def addto(accs, l, s):
                return tuple(
                    accs[k] + s * rows_v[l, pl.ds(k * NLANE, NLANE)]
                    for k in range(EMBED // NLANE))

            def group_body(g, accs):
                gbase = pl.multiple_of(g * NLANE, NLANE)
                wtg = wt_v[pl.ds(gbase, NLANE)]
                for j in range(NLANE):
                    accs = addto(accs, gbase + j, wtg[j])
                return accs

            accs = lax.fori_loop(
                0, L // NLANE, group_body,
                tuple(jnp.zeros((NLANE,), jnp.float32)
                      for _ in range(EMBED // NLANE)))
            gbase = (L // NLANE) * NLANE
            wtg = wt_v[pl.ds(gbase, NLANE)]
            for j in range(L % NLANE):
                accs = addto(accs, gbase + j, wtg[j])
            for j in range(EMBED // NLANE):
                pooled_v[i, pl.ds(j * NLANE, NLANE)] = accs[j]
            return 0

        lax.fori_loop(0, ROWS_PER_W, body, 0)
        pltpu.sync_copy(pooled_v, pooled_hbm.at[pl.ds(row0, ROWS_PER_W)])
        pltpu.sync_copy(len_v, len_hbm.at[pl.ds(row0, ROWS_PER_W)])

    return k(ids_flat, pairs, w)


def _head(pooled, counts, head_W, head_b):
    def hk(p_ref, l_ref, w_ref, b_ref, log_ref, enc_ref):
        length = jnp.sum(l_ref[...], axis=1, keepdims=True) + 1e-16
        p = p_ref[...] / length
        norm = jnp.sqrt(jnp.sum(p * p, axis=1, keepdims=True))
        enc = p / jnp.maximum(norm, 1e-12)
        enc_ref[...] = enc
        log_ref[...] = (
            jnp.dot(enc, w_ref[...], preferred_element_type=jnp.float32)
            + b_ref[...])

    return pl.pallas_call(
        hk,
        out_shape=(
            jax.ShapeDtypeStruct((B, OUT), jnp.float32),
            jax.ShapeDtypeStruct((B, EMBED), jnp.float32),
        ),
    )(pooled, counts, head_W, head_b)


def kernel(input_ids, vectors, w, head_W, head_b):
    ids_flat = input_ids.reshape(-1).astype(jnp.int32)
    pairs = _row_table(vectors.T)
    pooled, counts = _sc_pool(ids_flat, pairs, w)
    logits, encoded = _head(pooled, counts, head_W, head_b.reshape(1, OUT))
    return (logits, encoded)


# speedup vs baseline: 1.5531x; 1.2908x over previous
import functools

import jax
import jax.numpy as jnp
from jax import lax
from jax.experimental import pallas as pl
from jax.experimental.pallas import tpu as pltpu
from jax.experimental.pallas import tpu_sc as plsc

VOCAB = 1000000
EMBED = 64
B = 1024
L = 200
OUT = 2
PAD = 0

NC = 2
NS = 16
NW = NC * NS
ROWS_PER_W = B // NW
LPAD = 208
C0 = 128
C1 = L - C0
NLANE = 16


def _sc_pool(ids_flat, vectors, w):
    mesh = plsc.VectorSubcoreMesh(core_axis_name="c", subcore_axis_name="s")

    @functools.partial(
        pl.kernel,
        out_type=(
            jax.ShapeDtypeStruct((B, EMBED), jnp.float32),
            jax.ShapeDtypeStruct((B, NLANE), jnp.float32),
        ),
        mesh=mesh,
        compiler_params=pltpu.CompilerParams(use_tc_tiling_on_sc=True),
        scratch_types=[
            pltpu.VMEM((LPAD,), jnp.int32),
            pltpu.VMEM((LPAD,), jnp.float32),
            pltpu.VMEM((L, EMBED), jnp.float32),
            pltpu.VMEM((LPAD,), jnp.float32),
            pltpu.VMEM((ROWS_PER_W, EMBED), jnp.float32),
            pltpu.VMEM((ROWS_PER_W, NLANE), jnp.float32),
            pltpu.SemaphoreType.DMA,
            pltpu.SemaphoreType.DMA,
        ],
    )
    def k(ids_hbm, vec_hbm, w_hbm, pooled_hbm, len_hbm,
          idx_v, wv_v, rows_v, wt_v, pooled_v, len_v, sem, semr):
        wid = lax.axis_index("s") * NC + lax.axis_index("c")
        row0 = wid * ROWS_PER_W
        lanes = lax.iota(jnp.int32, NLANE)

        def body(i, _):
            row = row0 + i
            base = pl.multiple_of(row * L, 8)
            pltpu.sync_copy(ids_hbm.at[pl.ds(base, L)], idx_v.at[pl.ds(0, L)])
            cps = (
                pltpu.async_copy(w_hbm.at[idx_v.at[pl.ds(0, C0)]],
                                 wv_v.at[pl.ds(0, C0)], sem),
                pltpu.async_copy(w_hbm.at[idx_v.at[pl.ds(C0, C1)]],
                                 wv_v.at[pl.ds(C0, C1)], sem),
            )
            for g in range(L // NLANE):
                idg = idx_v[pl.ds(g * NLANE, NLANE)]
                for j in range(NLANE):
                    l = g * NLANE + j
                    pltpu.async_copy(vec_hbm.at[idg[j]], rows_v.at[l], semr)
            idg = idx_v[pl.ds((L // NLANE) * NLANE, NLANE)]
            for j in range(L % NLANE):
                l = (L // NLANE) * NLANE + j
                pltpu.async_copy(vec_hbm.at[idg[j]], rows_v.at[l], semr)

            for cp in cps:
                cp.wait()
            cnt = jnp.zeros((NLANE,), jnp.float32)
            for c in range(LPAD // NLANE):
                ids_c = idx_v[pl.ds(c * NLANE, NLANE)]
                wv_c = wv_v[pl.ds(c * NLANE, NLANE)]
                m = jnp.logical_and(lanes + (c * NLANE) < L, ids_c != PAD)
                sig = 1.0 / (1.0 + jnp.exp(-wv_c))
                wt_v[pl.ds(c * NLANE, NLANE)] = jnp.where(m, sig, 0.0)
                cnt = cnt + jnp.where(m, 1.0, 0.0)
            len_v[i, pl.ds(0, NLANE)] = cnt

            pltpu.make_async_copy(
                vec_hbm.at[pl.ds(0, L)], rows_v, semr).wait()

            def addto(accs, l, s):
                return tuple(
                    accs[k] + s * rows_v[l, pl.ds(k * NLANE, NLANE)]
                    for k in range(EMBED // NLANE))

            def group_body(g, accs):
                gbase = pl.multiple_of(g * NLANE, NLANE)
                wtg = wt_v[pl.ds(gbase, NLANE)]
                for j in range(NLANE):
                    accs = addto(accs, gbase + j, wtg[j])
                return accs

            accs = lax.fori_loop(
                0, L // NLANE, group_body,
                tuple(jnp.zeros((NLANE,), jnp.float32)
                      for _ in range(EMBED // NLANE)))
            gbase = (L // NLANE) * NLANE
            wtg = wt_v[pl.ds(gbase, NLANE)]
            for j in range(L % NLANE):
                accs = addto(accs, gbase + j, wtg[j])
            for j in range(EMBED // NLANE):
                pooled_v[i, pl.ds(j * NLANE, NLANE)] = accs[j]
            return 0

        lax.fori_loop(0, ROWS_PER_W, body, 0)
        pltpu.sync_copy(pooled_v, pooled_hbm.at[pl.ds(row0, ROWS_PER_W)])
        pltpu.sync_copy(len_v, len_hbm.at[pl.ds(row0, ROWS_PER_W)])

    return k(ids_flat, vectors, w)


def _head(pooled, counts, head_W, head_b):
    def hk(p_ref, l_ref, w_ref, b_ref, log_ref, enc_ref):
        length = jnp.sum(l_ref[...], axis=1, keepdims=True) + 1e-16
        p = p_ref[...] / length
        norm = jnp.sqrt(jnp.sum(p * p, axis=1, keepdims=True))
        enc = p / jnp.maximum(norm, 1e-12)
        enc_ref[...] = enc
        log_ref[...] = (
            jnp.dot(enc, w_ref[...], preferred_element_type=jnp.float32)
            + b_ref[...])

    return pl.pallas_call(
        hk,
        out_shape=(
            jax.ShapeDtypeStruct((B, OUT), jnp.float32),
            jax.ShapeDtypeStruct((B, EMBED), jnp.float32),
        ),
    )(pooled, counts, head_W, head_b)


def kernel(input_ids, vectors, w, head_W, head_b):
    ids_flat = input_ids.reshape(-1).astype(jnp.int32)
    pooled, counts = _sc_pool(ids_flat, vectors, w)
    logits, encoded = _head(pooled, counts, head_W, head_b.reshape(1, OUT))
    return (logits, encoded)
